# baseline (speedup 1.0000x reference)
import functools
import random

import jax
import jax.numpy as jnp
import numpy as np
from jax import lax
from jax.experimental import pallas as pl
from jax.experimental.pallas import tpu as pltpu
from jax.experimental.pallas import tpu_sc as plsc

_B = 16
_L = 4096
_SPAN = 8
_NCHUNK = _L // 16
_TPAD = _B * _L + 16

_rng = random.Random(0)
_TABLE = np.fromiter((_rng.randrange(1, 2 * _SPAN) for _ in range(_B * _L)),
                     dtype=np.int32, count=_B * _L)


def _iota():
    return lax.iota(jnp.int32, 16)


def _rgather(vec, idx):
    return vec.at[idx].get(mode="promise_in_bounds")


@functools.partial(
    pl.kernel,
    out_type=(jax.ShapeDtypeStruct((_B, _L), jnp.int32),
              jax.ShapeDtypeStruct((_B,), jnp.int32)),
    mesh=plsc.VectorSubcoreMesh(core_axis_name="c", subcore_axis_name="s"),
    scratch_types=[
        pltpu.VMEM((_TPAD,), jnp.int32),
        pltpu.VMEM((_L,), jnp.int32),
        pltpu.VMEM((_L,), jnp.int32),
        pltpu.VMEM((16,), jnp.int32),
    ],
)
def _chunker(rtm_hbm, table_hbm, seg_hbm, cnt_hbm,
             table_v, row_v, seg_v, cnt_v):
    cid = lax.axis_index("c")
    sid = lax.axis_index("s")

    @pl.when((cid == 0) & (sid == 0))
    def _work():
        pltpu.sync_copy(table_hbm, table_v.at[pl.ds(0, _B * _L)])
        lane = _iota()
        lane_m1 = jnp.maximum(lane - 1, 0)
        lane_p1 = jnp.minimum(lane + 1, 15)
        zeros = lane & 0
        firstlane = 1 - jnp.minimum(lane, 1)

        def row_body(b, carry):
            c0, nxt0, cnt_acc = carry
            pltpu.sync_copy(rtm_hbm.at[b], row_v)

            lv_vec = _rgather(row_v[pl.ds(_L - 16, 16)], zeros + 15)

            def pre_body(i, a):
                vals = row_v[pl.ds(i * 16, 16)]
                pos = lane + i * 16
                diff = jnp.minimum(vals ^ lv_vec, 1)
                return jnp.maximum(a, diff * (pos + 1) - 1)

            pacc = lax.fori_loop(0, _NCHUNK, pre_body, zeros - 1)
            for s in (8, 4, 2, 1):
                pacc = jnp.maximum(pacc, _rgather(pacc, (lane + s) & 15))
            fb_vec = lv_vec * _L + (1 - lv_vec) * jnp.maximum(pacc + 2, 1)

            def chunk_body(i, ch_carry):
                c_s, u_v, prev_last = ch_carry
                base = i * 16
                vals = row_v[pl.ds(base, 16)]
                pos = lane + base
                shifted = _rgather(vals, lane_m1)
                xor0 = jnp.where(i == 0, jnp.int32(1), vals[0] ^ prev_last)
                ns_i = (vals ^ shifted) | (firstlane * xor0)
                vld_i = jnp.clip(fb_vec - pos, 0, 1)
                code = ns_i * vld_i + vld_i
                tw = table_v[pl.ds(c_s, 16)]
                bits = zeros

                for j in range(16):
                    g = _rgather(code, zeros + j)
                    nsb = jnp.maximum(g - 1, 0)
                    vldb = jnp.minimum(g, 1)
                    peq = 1 - jnp.minimum(jnp.abs(u_v), 1)
                    consume = jnp.minimum(nsb + vldb * peq, 1)
                    d_v = _rgather(tw, zeros)
                    u_v = (u_v - 1) + consume * (d_v - u_v)
                    bits = bits + consume * (1 << j)
                    sh = _rgather(tw, lane_p1)
                    tw = tw + consume * (sh - tw)

                bits0 = _rgather(bits, zeros)
                bits_vec = jnp.right_shift(bits0, lane) & 1
                csum = bits_vec
                for s in (1, 2, 4, 8):
                    ind = jnp.clip(lane - s + 1, 0, 1)
                    csum = csum + _rgather(csum, jnp.maximum(lane - s, 0)) * ind
                c_incl = (zeros + (c_s - c0)) + csum
                seg_v[pl.ds(base, 16)] = vld_i * c_incl - 1
                c_next = c_s + csum[15]
                return (c_next, u_v, vals[15])

            c_end, u_end, _ = lax.fori_loop(
                0, _NCHUNK, chunk_body, (c0, nxt0, jnp.int32(0)))
            u_row_end = u_end + _L
            pltpu.sync_copy(seg_v, seg_hbm.at[b])
            eq_b = 1 - jnp.minimum(jnp.abs(lane - b), 1)
            cnt_acc = cnt_acc + eq_b * (c_end - c0)
            return (c_end, u_row_end, cnt_acc)

        init = (jnp.int32(0), zeros - 1, zeros)
        _, _, cnt_final = lax.fori_loop(0, _B, row_body, init)
        cnt_v[pl.ds(0, 16)] = cnt_final
        pltpu.sync_copy(cnt_v, cnt_hbm)


def kernel(inp, padding_mask, regular_tokens_mask):
    del inp, padding_mask
    table = jnp.asarray(_TABLE)
    seg_ids, n_chunks = _chunker(regular_tokens_mask.astype(jnp.int32), table)
    return (seg_ids, n_chunks)


# speedup vs baseline: 1.4811x; 1.4811x over previous
import functools
import random

import jax
import jax.numpy as jnp
import numpy as np
from jax import lax
from jax.experimental import pallas as pl
from jax.experimental.pallas import tpu as pltpu
from jax.experimental.pallas import tpu_sc as plsc

_B = 16
_L = 4096
_SPAN = 8
_NCHUNK = _L // 16
_TPAD = _B * _L + 64
_RING = 0
_FLAGS = 64

_rng = random.Random(0)
_TABLE = np.fromiter((_rng.randrange(1, 2 * _SPAN) for _ in range(_B * _L)),
                     dtype=np.int32, count=_B * _L)


def _iota():
    return lax.iota(jnp.int32, 16)


def _rgather(vec, idx):
    return vec.at[idx].get(mode="promise_in_bounds")


@functools.partial(
    pl.kernel,
    out_type=(jax.ShapeDtypeStruct((_B, _L), jnp.int32),
              jax.ShapeDtypeStruct((_B,), jnp.int32)),
    mesh=plsc.VectorSubcoreMesh(core_axis_name="c", subcore_axis_name="s"),
    scratch_types=[
        pltpu.VMEM((_TPAD,), jnp.int32),
        pltpu.VMEM((_L,), jnp.int32),
        pltpu.VMEM((_L,), jnp.int32),
        pltpu.VMEM((16,), jnp.int32),
        pltpu.SMEM((320,), jnp.int32),
    ],
)
def _chunker(rtm_hbm, table_hbm, seg_hbm, cnt_hbm,
             table_v, row_v, seg_v, cnt_v, sm):
    cid = lax.axis_index("c")
    sid = lax.axis_index("s")

    @pl.when((cid == 0) & (sid == 0))
    def _work():
        pltpu.sync_copy(table_hbm, table_v.at[pl.ds(0, _B * _L)])
        lane = _iota()
        lane_m1 = jnp.maximum(lane - 1, 0)
        zeros = lane & 0
        firstlane = 1 - jnp.minimum(lane, 1)
        eqmask = [1 - jnp.minimum(jnp.abs(lane - t), 1) for t in range(16)]

        def row_body(b, carry):
            c0, u0, cnt_acc = carry
            pltpu.sync_copy(rtm_hbm.at[b], row_v)

            lv_vec = _rgather(row_v[pl.ds(_L - 16, 16)], zeros + 15)

            def pre_group(g, gcarry):
                pacc_g, prev_vals = gcarry
                accum = zeros
                is_g0 = 1 - jnp.minimum(g, 1)
                pv = prev_vals
                for t in range(16):
                    basep = (g * 16 + t) * 16
                    vals = row_v[pl.ds(basep, 16)]
                    sh = _rgather(vals, lane_m1)
                    prev15 = _rgather(pv, zeros + 15)
                    if t == 0:
                        x0 = (vals ^ prev15) | (zeros + is_g0)
                    else:
                        x0 = vals ^ prev15
                    ns = (vals ^ sh) * (1 - firstlane) + firstlane * x0
                    word = jnp.left_shift(ns, lane)
                    for s in (8, 4, 2, 1):
                        word = word + _rgather(word, (lane + s) & 15)
                    accum = accum + eqmask[t] * word
                    pos = lane + basep
                    diff = jnp.minimum(vals ^ lv_vec, 1)
                    pacc_g = jnp.maximum(pacc_g, diff * (pos + 1) - 1)
                    pv = vals
                for t in range(16):
                    sm[_FLAGS + g * 16 + t] = accum[t]
                return (pacc_g, pv)

            pacc, _ = lax.fori_loop(0, 16, pre_group, (zeros - 1, zeros))
            for s in (8, 4, 2, 1):
                pacc = jnp.maximum(pacc, _rgather(pacc, (lane + s) & 15))
            fb_vec = lv_vec * _L + (1 - lv_vec) * jnp.maximum(pacc + 2, 1)
            fb_s = fb_vec[0]

            for w in range(3):
                tw = table_v[pl.ds(c0 + w * 16, 16)]
                for t in range(16):
                    sm[_RING + ((c0 + w * 16 + t) & 63)] = tw[t]

            def chunk_body(i, ch_carry):
                c_s, u_s = ch_carry
                base = i * 16
                c_in = c_s
                rv = table_v[pl.ds(c_in + 48, 16)]
                for t in range(16):
                    sm[_RING + ((c_in + 48 + t) & 63)] = rv[t]
                flagw = sm[_FLAGS + i]
                nv = jnp.clip(fb_s - base, 0, 16)
                cbits = jnp.int32(0)

                for j in range(16):
                    ns_j = (jnp.right_shift(flagw, j) & 1) != 0
                    vld_j = j < nv
                    consume = vld_j & (ns_j | (u_s == 0))
                    d = sm[_RING + (c_s & 63)]
                    u_s = jnp.where(consume, d, u_s) - 1
                    ci = jnp.where(consume, 1, 0)
                    c_s = c_s + ci
                    cbits = cbits + (ci << j)

                bits_vec = jnp.right_shift(zeros + cbits, lane) & 1
                csum = bits_vec
                for s in (1, 2, 4, 8):
                    ind = jnp.clip(lane - s + 1, 0, 1)
                    csum = csum + _rgather(csum, jnp.maximum(lane - s, 0)) * ind
                pos = lane + base
                vld_i = jnp.clip(fb_vec - pos, 0, 1)
                c_incl = (zeros + (c_in - c0)) + csum
                seg_v[pl.ds(base, 16)] = vld_i * c_incl - 1
                return (c_s, u_s)

            c_end, u_end = lax.fori_loop(0, _NCHUNK, chunk_body, (c0, u0))
            pltpu.sync_copy(seg_v, seg_hbm.at[b])
            eq_b = 1 - jnp.minimum(jnp.abs(lane - b), 1)
            cnt_acc = cnt_acc + eq_b * (c_end - c0)
            return (c_end, u_end + _L, cnt_acc)

        init = (jnp.int32(0), jnp.int32(-1), zeros)
        _, _, cnt_final = lax.fori_loop(0, _B, row_body, init)
        cnt_v[pl.ds(0, 16)] = cnt_final
        pltpu.sync_copy(cnt_v, cnt_hbm)


def kernel(inp, padding_mask, regular_tokens_mask):
    del inp, padding_mask
    table = jnp.asarray(_TABLE)
    seg_ids, n_chunks = _chunker(regular_tokens_mask.astype(jnp.int32), table)
    return (seg_ids, n_chunks)


# speedup vs baseline: 1.7102x; 1.1547x over previous
import functools
import random

import jax
import jax.numpy as jnp
import numpy as np
from jax import lax
from jax.experimental import pallas as pl
from jax.experimental.pallas import tpu as pltpu
from jax.experimental.pallas import tpu_sc as plsc

_B = 16
_L = 4096
_SPAN = 8
_NCHUNK = _L // 16
_TPAD = _B * _L + 64
_RING = 0
_FLAGS = 64

_rng = random.Random(0)
_TABLE = np.fromiter((_rng.randrange(1, 2 * _SPAN) for _ in range(_B * _L)),
                     dtype=np.int32, count=_B * _L)


def _iota():
    return lax.iota(jnp.int32, 16)


def _rgather(vec, idx):
    return vec.at[idx].get(mode="promise_in_bounds")


@functools.partial(
    pl.kernel,
    out_type=(jax.ShapeDtypeStruct((_B, _L), jnp.int32),
              jax.ShapeDtypeStruct((_B,), jnp.int32)),
    mesh=plsc.VectorSubcoreMesh(core_axis_name="c", subcore_axis_name="s"),
    scratch_types=[
        pltpu.VMEM((_TPAD,), jnp.int32),
        pltpu.VMEM((_L,), jnp.int32),
        pltpu.VMEM((_L,), jnp.int32),
        pltpu.VMEM((16,), jnp.int32),
        pltpu.SMEM((320,), jnp.int32),
    ],
)
def _chunker(rtm_hbm, table_hbm, seg_hbm, cnt_hbm,
             table_v, row_v, seg_v, cnt_v, sm):
    cid = lax.axis_index("c")
    sid = lax.axis_index("s")

    @pl.when((cid == 0) & (sid == 0))
    def _work():
        pltpu.sync_copy(table_hbm, table_v.at[pl.ds(0, _B * _L)])
        lane = _iota()
        lane_m1 = jnp.maximum(lane - 1, 0)
        zeros = lane & 0
        firstlane = 1 - jnp.minimum(lane, 1)
        eqmask = [1 - jnp.minimum(jnp.abs(lane - t), 1) for t in range(16)]

        def row_body(b, carry):
            c0, u0, cnt_acc = carry
            pltpu.sync_copy(rtm_hbm.at[b], row_v)

            lv_vec = _rgather(row_v[pl.ds(_L - 16, 16)], zeros + 15)

            def pre_group(g, gcarry):
                pacc_g, prev_vals = gcarry
                accum = zeros
                is_g0 = 1 - jnp.minimum(g, 1)
                pv = prev_vals
                for t in range(16):
                    basep = (g * 16 + t) * 16
                    vals = row_v[pl.ds(basep, 16)]
                    sh = _rgather(vals, lane_m1)
                    prev15 = _rgather(pv, zeros + 15)
                    if t == 0:
                        x0 = (vals ^ prev15) | (zeros + is_g0)
                    else:
                        x0 = vals ^ prev15
                    ns = (vals ^ sh) * (1 - firstlane) + firstlane * x0
                    word = jnp.left_shift(ns, lane)
                    for s in (8, 4, 2, 1):
                        word = word + _rgather(word, (lane + s) & 15)
                    accum = accum + eqmask[t] * word
                    pos = lane + basep
                    diff = jnp.minimum(vals ^ lv_vec, 1)
                    pacc_g = jnp.maximum(pacc_g, diff * (pos + 1) - 1)
                    pv = vals
                for t in range(16):
                    sm[_FLAGS + g * 16 + t] = accum[t]
                return (pacc_g, pv)

            pacc, _ = lax.fori_loop(0, 16, pre_group, (zeros - 1, zeros))
            for s in (8, 4, 2, 1):
                pacc = jnp.maximum(pacc, _rgather(pacc, (lane + s) & 15))
            fb_vec = lv_vec * _L + (1 - lv_vec) * jnp.maximum(pacc + 2, 1)
            fb_s = fb_vec[0]

            for w in range(3):
                tw = table_v[pl.ds(c0 + w * 16, 16)]
                for t in range(16):
                    sm[_RING + ((c0 + w * 16 + t) & 63)] = tw[t]

            nfull = jnp.right_shift(fb_s, 4)

            def seg_epilogue(base, c_in, cbits, mask_valid):
                bits_vec = jnp.right_shift(zeros + cbits, lane) & 1
                csum = bits_vec
                for s in (1, 2, 4, 8):
                    ind = jnp.clip(lane - s + 1, 0, 1)
                    csum = csum + _rgather(csum, jnp.maximum(lane - s, 0)) * ind
                c_incl = (zeros + (c_in - c0)) + csum
                if mask_valid:
                    pos = lane + base
                    vld_i = jnp.clip(fb_vec - pos, 0, 1)
                    seg_v[pl.ds(base, 16)] = vld_i * c_incl - 1
                else:
                    seg_v[pl.ds(base, 16)] = c_incl - 1

            def fast_body(i, ch_carry):
                c_s, u_s = ch_carry
                base = i * 16
                c_in = c_s
                rv = table_v[pl.ds(c_in + 48, 16)]
                for t in range(16):
                    sm[_RING + ((c_in + 48 + t) & 63)] = rv[t]
                flagw = sm[_FLAGS + i]
                cbits = jnp.int32(0)

                for j in range(16):
                    consume = ((jnp.right_shift(flagw, j) & 1) != 0) | (u_s == 0)
                    d = sm[_RING + (c_s & 63)]
                    u_s = jnp.where(consume, d, u_s) - 1
                    c_s = jnp.where(consume, c_s + 1, c_s)
                    cbits = jnp.where(consume, cbits | (1 << j), cbits)

                seg_epilogue(base, c_in, cbits, mask_valid=False)
                return (c_s, u_s)

            c_f, u_f = lax.fori_loop(0, nfull, fast_body, (c0, u0))

            def partial_body(args):
                c_s, u_s = args
                i = nfull
                base = i * 16
                c_in = c_s
                flagw = sm[_FLAGS + i]
                nv = fb_s - base
                cbits = jnp.int32(0)
                for j in range(16):
                    ns_j = (jnp.right_shift(flagw, j) & 1) != 0
                    vld_j = j < nv
                    consume = vld_j & (ns_j | (u_s == 0))
                    d = sm[_RING + (c_s & 63)]
                    u_s = jnp.where(consume, d, u_s) - 1
                    c_s = jnp.where(consume, c_s + 1, c_s)
                    cbits = jnp.where(consume, cbits | (1 << j), cbits)
                seg_epilogue(base, c_in, cbits, mask_valid=True)
                return (c_s, u_s)

            c_p, u_p = lax.cond(nfull < _NCHUNK, partial_body,
                                lambda args: args, (c_f, u_f))

            def inv_body(i, _):
                seg_v[pl.ds(i * 16, 16)] = zeros - 1
                return 0

            lax.fori_loop(nfull + 1, _NCHUNK, inv_body, 0)
            n_inv = jnp.maximum(_NCHUNK - 1 - nfull, 0)
            c_end = c_p
            u_end = u_p - 16 * n_inv
            pltpu.sync_copy(seg_v, seg_hbm.at[b])
            eq_b = 1 - jnp.minimum(jnp.abs(lane - b), 1)
            cnt_acc = cnt_acc + eq_b * (c_end - c0)
            return (c_end, u_end + _L, cnt_acc)

        init = (jnp.int32(0), jnp.int32(-1), zeros)
        _, _, cnt_final = lax.fori_loop(0, _B, row_body, init)
        cnt_v[pl.ds(0, 16)] = cnt_final
        pltpu.sync_copy(cnt_v, cnt_hbm)


def kernel(inp, padding_mask, regular_tokens_mask):
    del inp, padding_mask
    table = jnp.asarray(_TABLE)
    seg_ids, n_chunks = _chunker(regular_tokens_mask.astype(jnp.int32), table)
    return (seg_ids, n_chunks)


# speedup vs baseline: 2.0373x; 1.1913x over previous
import functools
import random

import jax
import jax.numpy as jnp
import numpy as np
from jax import lax
from jax.experimental import pallas as pl
from jax.experimental.pallas import tpu as pltpu
from jax.experimental.pallas import tpu_sc as plsc

_B = 16
_L = 4096
_SPAN = 8
_NCHUNK = _L // 16
_TPAD = _B * _L + 64
_RING = 0
_FLAGS = 64

_rng = random.Random(0)
_TABLE = np.fromiter((_rng.randrange(1, 2 * _SPAN) for _ in range(_B * _L)),
                     dtype=np.int32, count=_B * _L)


def _iota():
    return lax.iota(jnp.int32, 16)


def _rgather(vec, idx):
    return vec.at[idx].get(mode="promise_in_bounds")


@functools.partial(
    pl.kernel,
    out_type=(jax.ShapeDtypeStruct((_B, _L), jnp.int32),
              jax.ShapeDtypeStruct((_B,), jnp.int32)),
    mesh=plsc.VectorSubcoreMesh(core_axis_name="c", subcore_axis_name="s"),
    scratch_types=[
        pltpu.VMEM((_TPAD,), jnp.int32),
        pltpu.VMEM((_L,), jnp.int32),
        pltpu.VMEM((_L,), jnp.int32),
        pltpu.VMEM((_L,), jnp.int32),
        pltpu.VMEM((_L,), jnp.int32),
        pltpu.VMEM((16,), jnp.int32),
        pltpu.SMEM((320,), jnp.int32),
        pltpu.SemaphoreType.DMA,
        pltpu.SemaphoreType.DMA,
        pltpu.SemaphoreType.DMA,
        pltpu.SemaphoreType.DMA,
    ],
)
def _chunker(rtm_hbm, table_hbm, seg_hbm, cnt_hbm,
             table_v, row_a, row_b, seg_a, seg_b, cnt_v, sm,
             sin_a, sin_b, sout_a, sout_b):
    cid = lax.axis_index("c")
    sid = lax.axis_index("s")

    @pl.when((cid == 0) & (sid == 0))
    def _work():
        pltpu.sync_copy(table_hbm, table_v.at[pl.ds(0, _B * _L)])
        lane = _iota()
        lane_m1 = jnp.maximum(lane - 1, 0)
        zeros = lane & 0
        firstlane = 1 - jnp.minimum(lane, 1)
        eqmask = [1 - jnp.minimum(jnp.abs(lane - t), 1) for t in range(16)]

        def process_row(b, row_v, seg_v, carry):
            c0, u0, cnt_acc = carry

            lv_vec = _rgather(row_v[pl.ds(_L - 16, 16)], zeros + 15)

            def pre_group(g, gcarry):
                pacc_g, prev_vals = gcarry
                accum = zeros
                is_g0 = 1 - jnp.minimum(g, 1)
                pv = prev_vals
                for t in range(16):
                    basep = (g * 16 + t) * 16
                    vals = row_v[pl.ds(basep, 16)]
                    sh = _rgather(vals, lane_m1)
                    prev15 = _rgather(pv, zeros + 15)
                    if t == 0:
                        x0 = (vals ^ prev15) | (zeros + is_g0)
                    else:
                        x0 = vals ^ prev15
                    ns = (vals ^ sh) * (1 - firstlane) + firstlane * x0
                    word = jnp.left_shift(ns, lane)
                    for s in (8, 4, 2, 1):
                        word = word + _rgather(word, (lane + s) & 15)
                    accum = accum + eqmask[t] * word
                    pos = lane + basep
                    diff = jnp.minimum(vals ^ lv_vec, 1)
                    pacc_g = jnp.maximum(pacc_g, diff * (pos + 1) - 1)
                    pv = vals
                for t in range(16):
                    sm[_FLAGS + g * 16 + t] = accum[t]
                return (pacc_g, pv)

            pacc, _ = lax.fori_loop(0, 16, pre_group, (zeros - 1, zeros))
            for s in (8, 4, 2, 1):
                pacc = jnp.maximum(pacc, _rgather(pacc, (lane + s) & 15))
            fb_vec = lv_vec * _L + (1 - lv_vec) * jnp.maximum(pacc + 2, 1)
            fb_s = fb_vec[0]

            for w in range(3):
                tw = table_v[pl.ds(c0 + w * 16, 16)]
                for t in range(16):
                    sm[_RING + ((c0 + w * 16 + t) & 63)] = tw[t]

            nfull = jnp.right_shift(fb_s, 4)

            def seg_epilogue(base, c_in, cbits, mask_valid):
                bits_vec = jnp.right_shift(zeros + cbits, lane) & 1
                csum = bits_vec
                for s in (1, 2, 4, 8):
                    ind = jnp.clip(lane - s + 1, 0, 1)
                    csum = csum + _rgather(csum, jnp.maximum(lane - s, 0)) * ind
                c_incl = (zeros + (c_in - c0)) + csum
                if mask_valid:
                    pos = lane + base
                    vld_i = jnp.clip(fb_vec - pos, 0, 1)
                    seg_v[pl.ds(base, 16)] = vld_i * c_incl - 1
                else:
                    seg_v[pl.ds(base, 16)] = c_incl - 1

            def fast_body(i, ch_carry):
                c_s, u_s = ch_carry
                base = i * 16
                c_in = c_s
                flagw = sm[_FLAGS + i]
                cbits = jnp.int32(0)

                for j in range(16):
                    consume = ((jnp.right_shift(flagw, j) & 1) != 0) | (u_s == 0)
                    d = sm[_RING + (c_s & 63)]
                    u_s = jnp.where(consume, d, u_s) - 1
                    c_s = jnp.where(consume, c_s + 1, c_s)
                    cbits = jnp.where(consume, cbits | (1 << j), cbits)

                rv = table_v[pl.ds(c_in + 48, 16)]
                for t in range(16):
                    sm[_RING + ((c_in + 48 + t) & 63)] = rv[t]
                seg_epilogue(base, c_in, cbits, mask_valid=False)
                return (c_s, u_s)

            c_f, u_f = lax.fori_loop(0, nfull, fast_body, (c0, u0))

            def partial_body(args):
                c_s, u_s = args
                i = nfull
                base = i * 16
                c_in = c_s
                flagw = sm[_FLAGS + i]
                nv = fb_s - base
                cbits = jnp.int32(0)
                for j in range(16):
                    ns_j = (jnp.right_shift(flagw, j) & 1) != 0
                    vld_j = j < nv
                    consume = vld_j & (ns_j | (u_s == 0))
                    d = sm[_RING + (c_s & 63)]
                    u_s = jnp.where(consume, d, u_s) - 1
                    c_s = jnp.where(consume, c_s + 1, c_s)
                    cbits = jnp.where(consume, cbits | (1 << j), cbits)
                seg_epilogue(base, c_in, cbits, mask_valid=True)
                return (c_s, u_s)

            c_p, u_p = lax.cond(nfull < _NCHUNK, partial_body,
                                lambda args: args, (c_f, u_f))

            def inv_body(i, _):
                seg_v[pl.ds(i * 16, 16)] = zeros - 1
                return 0

            lax.fori_loop(nfull + 1, _NCHUNK, inv_body, 0)
            n_inv = jnp.maximum(_NCHUNK - 1 - nfull, 0)
            c_end = c_p
            u_end = u_p - 16 * n_inv

            eq_b = 1 - jnp.minimum(jnp.abs(lane - b), 1)
            cnt_acc = cnt_acc + eq_b * (c_end - c0)
            return (c_end, u_end + _L, cnt_acc)

        pltpu.async_copy(rtm_hbm.at[0], row_a, sin_a)

        def pair_body(pb, carry):
            for par, (rbuf, sbuf, sin, sout) in enumerate(
                    ((row_a, seg_a, sin_a, sout_a),
                     (row_b, seg_b, sin_b, sout_b))):
                b = pb * 2 + par
                pltpu.make_async_copy(rtm_hbm.at[b], rbuf, sin).wait()

                @pl.when(b + 1 < _B)
                def _prefetch():
                    nxt_buf = row_b if par == 0 else row_a
                    nxt_sin = sin_b if par == 0 else sin_a
                    pltpu.async_copy(rtm_hbm.at[b + 1], nxt_buf, nxt_sin)

                @pl.when(b >= 2)
                def _drain_seg():
                    pltpu.make_async_copy(sbuf, seg_hbm.at[b], sout).wait()

                carry = process_row(b, rbuf, sbuf, carry)
                pltpu.async_copy(sbuf, seg_hbm.at[b], sout)
            return carry

        init = (jnp.int32(0), jnp.int32(-1), zeros)
        _, _, cnt_final = lax.fori_loop(0, _B // 2, pair_body, init)
        pltpu.make_async_copy(seg_a, seg_hbm.at[_B - 2], sout_a).wait()
        pltpu.make_async_copy(seg_b, seg_hbm.at[_B - 1], sout_b).wait()
        cnt_v[pl.ds(0, 16)] = cnt_final
        pltpu.sync_copy(cnt_v, cnt_hbm)


def kernel(inp, padding_mask, regular_tokens_mask):
    del inp, padding_mask
    table = jnp.asarray(_TABLE)
    seg_ids, n_chunks = _chunker(regular_tokens_mask.astype(jnp.int32), table)
    return (seg_ids, n_chunks)


# speedup vs baseline: 2.1461x; 1.0534x over previous
import functools
import random

import jax
import jax.numpy as jnp
import numpy as np
from jax import lax
from jax.experimental import pallas as pl
from jax.experimental.pallas import tpu as pltpu
from jax.experimental.pallas import tpu_sc as plsc

_B = 16
_L = 4096
_SPAN = 8
_NCHUNK = _L // 16
_TPAD = _B * _L + 64
_RING = 0
_FLAGS = 64

_rng = random.Random(0)
_TABLE = np.fromiter((_rng.randrange(1, 2 * _SPAN) for _ in range(_B * _L)),
                     dtype=np.int32, count=_B * _L)


def _iota():
    return lax.iota(jnp.int32, 16)


def _rgather(vec, idx):
    return vec.at[idx].get(mode="promise_in_bounds")


@functools.partial(
    pl.kernel,
    out_type=(jax.ShapeDtypeStruct((_B, _L), jnp.int32),
              jax.ShapeDtypeStruct((_B,), jnp.int32)),
    mesh=plsc.VectorSubcoreMesh(core_axis_name="c", subcore_axis_name="s"),
    scratch_types=[
        pltpu.VMEM((_TPAD,), jnp.int32),
        pltpu.VMEM((_L,), jnp.int32),
        pltpu.VMEM((_L,), jnp.int32),
        pltpu.VMEM((_L,), jnp.int32),
        pltpu.VMEM((_L,), jnp.int32),
        pltpu.VMEM((16,), jnp.int32),
        pltpu.SMEM((320,), jnp.int32),
        pltpu.SemaphoreType.DMA,
        pltpu.SemaphoreType.DMA,
        pltpu.SemaphoreType.DMA,
        pltpu.SemaphoreType.DMA,
    ],
)
def _chunker(rtm_hbm, table_hbm, seg_hbm, cnt_hbm,
             table_v, row_a, row_b, seg_a, seg_b, cnt_v, sm,
             sin_a, sin_b, sout_a, sout_b):
    cid = lax.axis_index("c")
    sid = lax.axis_index("s")

    @pl.when((cid == 0) & (sid == 0))
    def _work():
        pltpu.sync_copy(table_hbm, table_v.at[pl.ds(0, _B * _L)])
        lane = _iota()
        lane_m1 = jnp.maximum(lane - 1, 0)
        zeros = lane & 0
        firstlane = 1 - jnp.minimum(lane, 1)
        eqmask = [1 - jnp.minimum(jnp.abs(lane - t), 1) for t in range(16)]

        def process_row(b, row_v, seg_v, carry):
            c0, u0, cnt_acc = carry

            lv_vec = _rgather(row_v[pl.ds(_L - 16, 16)], zeros + 15)

            def pre_group(g, gcarry):
                pacc_g, prev_vals = gcarry
                accum = zeros
                is_g0 = 1 - jnp.minimum(g, 1)
                pv = prev_vals
                for t in range(16):
                    basep = (g * 16 + t) * 16
                    vals = row_v[pl.ds(basep, 16)]
                    sh = _rgather(vals, lane_m1)
                    prev15 = _rgather(pv, zeros + 15)
                    if t == 0:
                        x0 = (vals ^ prev15) | (zeros + is_g0)
                    else:
                        x0 = vals ^ prev15
                    ns = (vals ^ sh) * (1 - firstlane) + firstlane * x0
                    word = jnp.left_shift(ns, 31 - lane)
                    for s in (8, 4, 2, 1):
                        word = word + _rgather(word, (lane + s) & 15)
                    accum = accum + eqmask[t] * word
                    pos = lane + basep
                    diff = jnp.minimum(vals ^ lv_vec, 1)
                    pacc_g = jnp.maximum(pacc_g, diff * (pos + 1) - 1)
                    pv = vals
                for t in range(16):
                    sm[_FLAGS + g * 16 + t] = accum[t]
                return (pacc_g, pv)

            pacc, _ = lax.fori_loop(0, 16, pre_group, (zeros - 1, zeros))
            for s in (8, 4, 2, 1):
                pacc = jnp.maximum(pacc, _rgather(pacc, (lane + s) & 15))
            fb_vec = lv_vec * _L + (1 - lv_vec) * jnp.maximum(pacc + 2, 1)
            fb_s = fb_vec[0]

            for w in range(3):
                tw = table_v[pl.ds(c0 + w * 16, 16)]
                for t in range(16):
                    sm[_RING + ((c0 + w * 16 + t) & 63)] = tw[t]

            nfull = jnp.right_shift(fb_s, 4)

            def seg_epilogue(base, c_in, cbits, mask_valid):
                bits_vec = jnp.right_shift(zeros + cbits, lane) & 1
                csum = bits_vec
                for s in (1, 2, 4, 8):
                    ind = jnp.clip(lane - s + 1, 0, 1)
                    csum = csum + _rgather(csum, jnp.maximum(lane - s, 0)) * ind
                c_incl = (zeros + (c_in - c0)) + csum
                if mask_valid:
                    pos = lane + base
                    vld_i = jnp.clip(fb_vec - pos, 0, 1)
                    seg_v[pl.ds(base, 16)] = vld_i * c_incl - 1
                else:
                    seg_v[pl.ds(base, 16)] = c_incl - 1

            def fast_body(i, ch_carry):
                c_s, u_s = ch_carry
                base = i * 16
                c_in = c_s
                flagw = sm[_FLAGS + i]
                cbits = jnp.int32(0)

                for j in range(16):
                    consume = (flagw < 0) | (u_s == 0)
                    flagw = jnp.left_shift(flagw, 1)
                    d = sm[_RING + (c_s & 63)]
                    u_s = jnp.where(consume, d, u_s) - 1
                    c_s = jnp.where(consume, c_s + 1, c_s)
                    cbits = jnp.where(consume, cbits | (1 << j), cbits)

                rv = table_v[pl.ds(c_in + 48, 16)]
                for t in range(16):
                    sm[_RING + ((c_in + 48 + t) & 63)] = rv[t]
                seg_epilogue(base, c_in, cbits, mask_valid=False)
                return (c_s, u_s)

            c_f, u_f = lax.fori_loop(0, nfull, fast_body, (c0, u0))

            def partial_body(args):
                c_s, u_s = args
                i = nfull
                base = i * 16
                c_in = c_s
                flagw = sm[_FLAGS + i]
                nv = fb_s - base
                cbits = jnp.int32(0)
                for j in range(16):
                    ns_j = flagw < 0
                    flagw = jnp.left_shift(flagw, 1)
                    vld_j = j < nv
                    consume = vld_j & (ns_j | (u_s == 0))
                    d = sm[_RING + (c_s & 63)]
                    u_s = jnp.where(consume, d, u_s) - 1
                    c_s = jnp.where(consume, c_s + 1, c_s)
                    cbits = jnp.where(consume, cbits | (1 << j), cbits)
                seg_epilogue(base, c_in, cbits, mask_valid=True)
                return (c_s, u_s)

            c_p, u_p = lax.cond(nfull < _NCHUNK, partial_body,
                                lambda args: args, (c_f, u_f))

            def inv_body(i, _):
                seg_v[pl.ds(i * 16, 16)] = zeros - 1
                return 0

            lax.fori_loop(nfull + 1, _NCHUNK, inv_body, 0)
            n_inv = jnp.maximum(_NCHUNK - 1 - nfull, 0)
            c_end = c_p
            u_end = u_p - 16 * n_inv

            eq_b = 1 - jnp.minimum(jnp.abs(lane - b), 1)
            cnt_acc = cnt_acc + eq_b * (c_end - c0)
            return (c_end, u_end + _L, cnt_acc)

        pltpu.async_copy(rtm_hbm.at[0], row_a, sin_a)

        def pair_body(pb, carry):
            for par, (rbuf, sbuf, sin, sout) in enumerate(
                    ((row_a, seg_a, sin_a, sout_a),
                     (row_b, seg_b, sin_b, sout_b))):
                b = pb * 2 + par
                pltpu.make_async_copy(rtm_hbm.at[b], rbuf, sin).wait()

                @pl.when(b + 1 < _B)
                def _prefetch():
                    nxt_buf = row_b if par == 0 else row_a
                    nxt_sin = sin_b if par == 0 else sin_a
                    pltpu.async_copy(rtm_hbm.at[b + 1], nxt_buf, nxt_sin)

                @pl.when(b >= 2)
                def _drain_seg():
                    pltpu.make_async_copy(sbuf, seg_hbm.at[b], sout).wait()

                carry = process_row(b, rbuf, sbuf, carry)
                pltpu.async_copy(sbuf, seg_hbm.at[b], sout)
            return carry

        init = (jnp.int32(0), jnp.int32(-1), zeros)
        _, _, cnt_final = lax.fori_loop(0, _B // 2, pair_body, init)
        pltpu.make_async_copy(seg_a, seg_hbm.at[_B - 2], sout_a).wait()
        pltpu.make_async_copy(seg_b, seg_hbm.at[_B - 1], sout_b).wait()
        cnt_v[pl.ds(0, 16)] = cnt_final
        pltpu.sync_copy(cnt_v, cnt_hbm)


def kernel(inp, padding_mask, regular_tokens_mask):
    del inp, padding_mask
    table = jnp.asarray(_TABLE)
    seg_ids, n_chunks = _chunker(regular_tokens_mask.astype(jnp.int32), table)
    return (seg_ids, n_chunks)
